# Initial kernel scaffold; baseline (speedup 1.0000x reference)
import jax
import jax.numpy as jnp
from jax.experimental import pallas as pl


def kernel(x, edge_index, W1, b1, W2, b2, W3, b3, W4, b4):
    raise NotImplementedError("write your pallas kernel here")


# speedup vs baseline: 10.9755x; 10.9755x over previous
import functools

import jax
import jax.numpy as jnp
from jax import lax
from jax.experimental import pallas as pl
from jax.experimental.pallas import tpu as pltpu
from jax.experimental.pallas import tpu_sc as plsc

_N = 10000
_E = 320000
_D = 128
_NHID = 32
_LAT = 16

_NP = 10240
_NTILES = 32
_BLK = 128
_NBLK = 80
_EPT = _BLK * _NBLK
_EP = _NTILES * _EPT
_RPT = _NP // 16
_BR = 1024


def _sc_mesh():
    return plsc.VectorSubcoreMesh(core_axis_name="c", subcore_axis_name="s")


def _make_sc_accum(F):
    @functools.partial(
        pl.kernel,
        out_type=jax.ShapeDtypeStruct((2, _NP, F), jnp.float32),
        mesh=_sc_mesh(),
        compiler_params=pltpu.CompilerParams(use_tc_tiling_on_sc=False),
        scratch_types=[
            pltpu.VMEM((_NBLK, _BLK), jnp.int32),
            pltpu.VMEM((_NBLK, _BLK), jnp.int32),
            pltpu.VMEM((_BLK, F), jnp.float32),
            pltpu.VMEM_SHARED((_NP, F), jnp.float32),
            pltpu.SemaphoreType.DMA,
            pltpu.SemaphoreType.DMA,
        ],
    )
    def k(src_hbm, dst_hbm, tbl_hbm, zeros_hbm, out_hbm,
          sidx, didx, rows, acc, gsem, ssem):
        c = lax.axis_index("c")
        s = lax.axis_index("s")
        w = s * 2 + c
        pltpu.sync_copy(src_hbm.at[w], sidx)
        pltpu.sync_copy(dst_hbm.at[w], didx)
        r0 = s * _RPT
        pltpu.sync_copy(zeros_hbm.at[pl.ds(r0, _RPT)], acc.at[pl.ds(r0, _RPT)])
        plsc.subcore_barrier()

        @pl.loop(0, _NBLK)
        def _(j):
            pltpu.async_copy(tbl_hbm.at[didx.at[j]], rows, gsem).wait()
            pltpu.async_copy(rows, acc.at[sidx.at[j]], ssem, add=True).wait()

        plsc.subcore_barrier()
        pltpu.sync_copy(acc.at[pl.ds(r0, _RPT)], out_hbm.at[c].at[pl.ds(r0, _RPT)])

    return k


def _make_sc_degree():
    @functools.partial(
        pl.kernel,
        out_type=(jax.ShapeDtypeStruct((2, _NP, 8), jnp.float32),
                  jax.ShapeDtypeStruct((2, _NP, 8), jnp.float32)),
        mesh=_sc_mesh(),
        compiler_params=pltpu.CompilerParams(use_tc_tiling_on_sc=False),
        scratch_types=[
            pltpu.VMEM((_NBLK, _BLK), jnp.int32),
            pltpu.VMEM((_NBLK, _BLK), jnp.int32),
            pltpu.VMEM((_BLK, 8), jnp.float32),
            pltpu.VMEM_SHARED((_NP, 8), jnp.float32),
            pltpu.VMEM_SHARED((_NP, 8), jnp.float32),
            pltpu.SemaphoreType.DMA,
        ],
    )
    def k(src_hbm, self_hbm, ones_hbm, zeros_hbm, outd_hbm, outs_hbm,
          sidx, fidx, ones, accd, accs, ssem):
        c = lax.axis_index("c")
        s = lax.axis_index("s")
        w = s * 2 + c
        pltpu.sync_copy(src_hbm.at[w], sidx)
        pltpu.sync_copy(self_hbm.at[w], fidx)
        pltpu.sync_copy(ones_hbm, ones)
        r0 = s * _RPT
        pltpu.sync_copy(zeros_hbm.at[pl.ds(r0, _RPT)], accd.at[pl.ds(r0, _RPT)])
        pltpu.sync_copy(zeros_hbm.at[pl.ds(r0, _RPT)], accs.at[pl.ds(r0, _RPT)])
        plsc.subcore_barrier()

        @pl.loop(0, _NBLK)
        def _(j):
            pltpu.async_copy(ones, accd.at[sidx.at[j]], ssem, add=True).wait()
            pltpu.async_copy(ones, accs.at[fidx.at[j]], ssem, add=True).wait()

        plsc.subcore_barrier()
        pltpu.sync_copy(accd.at[pl.ds(r0, _RPT)], outd_hbm.at[c].at[pl.ds(r0, _RPT)])
        pltpu.sync_copy(accs.at[pl.ds(r0, _RPT)], outs_hbm.at[c].at[pl.ds(r0, _RPT)])

    return k


def _tc_prep(degp, selfp, xp, W1, b1):
    grid = (_NP // _BR,)

    def body(degp_ref, selfp_ref, x_ref, w_ref, b_ref, dinv8_ref, g1_ref):
        deg = degp_ref[0, :, 0:1] + degp_ref[1, :, 0:1] + 1.0
        cnt = selfp_ref[0, :, 0:1] + selfp_ref[1, :, 0:1]
        dinv = lax.rsqrt(deg)
        coef = 2.0 + 3.0 * cnt
        cols = lax.broadcasted_iota(jnp.int32, (_BR, 8), 1)
        dinv8_ref[...] = jnp.where(
            cols == 0, jnp.broadcast_to(dinv, (_BR, 8)),
            jnp.where(cols == 1, jnp.broadcast_to(coef, (_BR, 8)), 0.0))
        g = jnp.dot(x_ref[...], w_ref[...],
                    preferred_element_type=jnp.float32) + b_ref[...]
        g1_ref[...] = dinv * g

    return pl.pallas_call(
        body,
        grid=grid,
        in_specs=[
            pl.BlockSpec((2, _BR, 8), lambda i: (0, i, 0)),
            pl.BlockSpec((2, _BR, 8), lambda i: (0, i, 0)),
            pl.BlockSpec((_BR, _D), lambda i: (i, 0)),
            pl.BlockSpec((_D, _NHID), lambda i: (0, 0)),
            pl.BlockSpec((1, _NHID), lambda i: (0, 0)),
        ],
        out_specs=[
            pl.BlockSpec((_BR, 8), lambda i: (i, 0)),
            pl.BlockSpec((_BR, _NHID), lambda i: (i, 0)),
        ],
        out_shape=[
            jax.ShapeDtypeStruct((_NP, 8), jnp.float32),
            jax.ShapeDtypeStruct((_NP, _NHID), jnp.float32),
        ],
    )(degp, selfp, xp, W1, b1)


def _tc_layer(accp, gp, dinv8, W, b, lap):
    F = gp.shape[1]
    F2 = W.shape[1]
    grid = (_NP // _BR,)

    def body(accp_ref, gp_ref, dinv8_ref, w_ref, b_ref, h_ref, gn_ref):
        dinv = dinv8_ref[:, 0:1]
        accsum = accp_ref[0] + accp_ref[1]
        if lap:
            coef = dinv8_ref[:, 1:2]
            h = jnp.maximum(coef * dinv * gp_ref[...] - dinv * accsum, 0.0)
        else:
            h = jnp.maximum(dinv * (accsum + gp_ref[...]), 0.0)
        h_ref[...] = h
        gn = jnp.dot(h, w_ref[...], preferred_element_type=jnp.float32) + b_ref[...]
        gn_ref[...] = dinv * gn

    return pl.pallas_call(
        body,
        grid=grid,
        in_specs=[
            pl.BlockSpec((2, _BR, F), lambda i: (0, i, 0)),
            pl.BlockSpec((_BR, F), lambda i: (i, 0)),
            pl.BlockSpec((_BR, 8), lambda i: (i, 0)),
            pl.BlockSpec((F, F2), lambda i: (0, 0)),
            pl.BlockSpec((1, F2), lambda i: (0, 0)),
        ],
        out_specs=[
            pl.BlockSpec((_BR, F), lambda i: (i, 0)),
            pl.BlockSpec((_BR, F2), lambda i: (i, 0)),
        ],
        out_shape=[
            jax.ShapeDtypeStruct((_NP, F), jnp.float32),
            jax.ShapeDtypeStruct((_NP, F2), jnp.float32),
        ],
    )(accp, gp, dinv8, W, b)


def _tc_final(accp, gp, dinv8):
    F = gp.shape[1]
    grid = (_NP // _BR,)

    def body(accp_ref, gp_ref, dinv8_ref, out_ref):
        dinv = dinv8_ref[:, 0:1]
        coef = dinv8_ref[:, 1:2]
        accsum = accp_ref[0] + accp_ref[1]
        out_ref[...] = jnp.maximum(coef * dinv * gp_ref[...] - dinv * accsum, 0.0)

    return pl.pallas_call(
        body,
        grid=grid,
        in_specs=[
            pl.BlockSpec((2, _BR, F), lambda i: (0, i, 0)),
            pl.BlockSpec((_BR, F), lambda i: (i, 0)),
            pl.BlockSpec((_BR, 8), lambda i: (i, 0)),
        ],
        out_specs=pl.BlockSpec((_BR, F), lambda i: (i, 0)),
        out_shape=jax.ShapeDtypeStruct((_NP, F), jnp.float32),
    )(accp, gp, dinv8)


@jax.jit
def kernel(x, edge_index, W1, b1, W2, b2, W3, b3, W4, b4):
    src = edge_index[0]
    dst = edge_index[1]
    pad = jnp.full((_EP - _E,), _N, jnp.int32)
    srcp = jnp.concatenate([src, pad]).reshape(_NTILES, _NBLK, _BLK)
    dstp = jnp.concatenate([dst, pad]).reshape(_NTILES, _NBLK, _BLK)
    selfi = jnp.where(src == dst, src, _N)
    selfp = jnp.concatenate([selfi, pad]).reshape(_NTILES, _NBLK, _BLK)
    xp = jnp.concatenate([x, jnp.zeros((_NP - _N, _D), jnp.float32)])

    ones8 = jnp.ones((_BLK, 8), jnp.float32)
    zeros8 = jnp.zeros((_NP, 8), jnp.float32)
    zeros16 = jnp.zeros((_NP, _LAT), jnp.float32)
    zeros32 = jnp.zeros((_NP, _NHID), jnp.float32)
    zeros128 = jnp.zeros((_NP, _D), jnp.float32)

    degp, sfp = _make_sc_degree()(srcp, selfp, ones8, zeros8)
    dinv8, g1p = _tc_prep(degp, sfp, xp, W1, b1.reshape(1, -1))

    a1 = _make_sc_accum(_NHID)(srcp, dstp, g1p, zeros32)
    _, g2p = _tc_layer(a1, g1p, dinv8, W2, b2.reshape(1, -1), lap=False)

    a2 = _make_sc_accum(_LAT)(srcp, dstp, g2p, zeros16)
    z, g3p = _tc_layer(a2, g2p, dinv8, W3, b3.reshape(1, -1), lap=False)

    a3 = _make_sc_accum(_NHID)(srcp, dstp, g3p, zeros32)
    _, g4p = _tc_layer(a3, g3p, dinv8, W4, b4.reshape(1, -1), lap=True)

    a4 = _make_sc_accum(_D)(srcp, dstp, g4p, zeros128)
    xrec = _tc_final(a4, g4p, dinv8)

    return xrec[:_N], z[:_N]


# speedup vs baseline: 13.3834x; 1.2194x over previous
import functools

import jax
import jax.numpy as jnp
from jax import lax
from jax.experimental import pallas as pl
from jax.experimental.pallas import tpu as pltpu
from jax.experimental.pallas import tpu_sc as plsc

_N = 10000
_E = 320000
_D = 128
_NHID = 32
_LAT = 16

_NP = 10240
_NTILES = 32
_BLK = 128
_NBLK = 80
_EPT = _BLK * _NBLK
_EP = _NTILES * _EPT
_RPT = _NP // 16
_BR = 1024
_DEG_LAG = 8


def _sc_mesh():
    return plsc.VectorSubcoreMesh(core_axis_name="c", subcore_axis_name="s",
                                  num_cores=2, num_subcores=16)


def _make_sc_accum(F, nbuf):
    pf = nbuf // 2

    @functools.partial(
        pl.kernel,
        out_type=jax.ShapeDtypeStruct((2, _NP, F), jnp.float32),
        mesh=_sc_mesh(),
        compiler_params=pltpu.CompilerParams(use_tc_tiling_on_sc=False),
        scratch_types=[
            pltpu.VMEM((_NBLK, _BLK), jnp.int32),
            pltpu.VMEM((_NBLK, _BLK), jnp.int32),
            [pltpu.VMEM((_BLK, F), jnp.float32) for _ in range(nbuf)],
            pltpu.VMEM_SHARED((_NP, F), jnp.float32),
            [pltpu.SemaphoreType.DMA for _ in range(nbuf)],
            [pltpu.SemaphoreType.DMA for _ in range(nbuf)],
        ],
    )
    def k(src_hbm, dst_hbm, tbl_hbm, zeros_hbm, out_hbm,
          sidx, didx, rows, acc, gsems, ssems):
        c = lax.axis_index("c")
        s = lax.axis_index("s")
        w = s * 2 + c
        pltpu.sync_copy(src_hbm.at[w], sidx)
        pltpu.sync_copy(dst_hbm.at[w], didx)
        r0 = s * _RPT
        pltpu.sync_copy(zeros_hbm.at[pl.ds(r0, _RPT)], acc.at[pl.ds(r0, _RPT)])
        plsc.subcore_barrier()

        gd = [None] * _NBLK
        sd = [None] * _NBLK
        s_waited = [False] * _NBLK

        def fire_gather(m):
            b = m % nbuf
            gd[m] = pltpu.async_copy(tbl_hbm.at[didx.at[m]], rows[b], gsems[b])

        for m in range(pf):
            fire_gather(m)
        for j in range(_NBLK):
            b = j % nbuf
            m = j + pf
            if m < _NBLK:
                p = m - nbuf
                if p >= 0:
                    sd[p].wait()
                    s_waited[p] = True
                fire_gather(m)
            gd[j].wait()
            sd[j] = pltpu.async_copy(rows[b], acc.at[sidx.at[j]], ssems[b],
                                     add=True)
        for j in range(_NBLK):
            if not s_waited[j]:
                sd[j].wait()

        plsc.subcore_barrier()
        pltpu.sync_copy(acc.at[pl.ds(r0, _RPT)], out_hbm.at[c].at[pl.ds(r0, _RPT)])

    return k


def _make_sc_accum_split(F, nbuf):
    Fh = F // 2
    nblk = 2 * _NBLK
    pf = nbuf // 2

    @functools.partial(
        pl.kernel,
        out_type=jax.ShapeDtypeStruct((2, _NP, Fh), jnp.float32),
        mesh=_sc_mesh(),
        compiler_params=pltpu.CompilerParams(use_tc_tiling_on_sc=False),
        scratch_types=[
            pltpu.VMEM((nblk, _BLK), jnp.int32),
            pltpu.VMEM((nblk, _BLK), jnp.int32),
            [pltpu.VMEM((_BLK, Fh), jnp.float32) for _ in range(nbuf)],
            pltpu.VMEM_SHARED((_NP, Fh), jnp.float32),
            [pltpu.SemaphoreType.DMA for _ in range(nbuf)],
            [pltpu.SemaphoreType.DMA for _ in range(nbuf)],
        ],
    )
    def k(src_hbm, dst_hbm, tblh_hbm, zeros_hbm, out_hbm,
          sidx, didx, rows, acc, gsems, ssems):
        c = lax.axis_index("c")
        s = lax.axis_index("s")
        pltpu.sync_copy(src_hbm.at[s], sidx)
        pltpu.sync_copy(dst_hbm.at[s], didx)
        tbl = tblh_hbm.at[c]
        r0 = s * _RPT
        pltpu.sync_copy(zeros_hbm.at[pl.ds(r0, _RPT)], acc.at[pl.ds(r0, _RPT)])
        plsc.subcore_barrier()

        gd = [None] * nblk
        sd = [None] * nblk
        s_waited = [False] * nblk

        def fire_gather(m):
            b = m % nbuf
            gd[m] = pltpu.async_copy(tbl.at[didx.at[m]], rows[b], gsems[b])

        for m in range(pf):
            fire_gather(m)
        for j in range(nblk):
            b = j % nbuf
            m = j + pf
            if m < nblk:
                p = m - nbuf
                if p >= 0:
                    sd[p].wait()
                    s_waited[p] = True
                fire_gather(m)
            gd[j].wait()
            sd[j] = pltpu.async_copy(rows[b], acc.at[sidx.at[j]], ssems[b],
                                     add=True)
        for j in range(nblk):
            if not s_waited[j]:
                sd[j].wait()

        plsc.subcore_barrier()
        pltpu.sync_copy(acc.at[pl.ds(r0, _RPT)], out_hbm.at[c].at[pl.ds(r0, _RPT)])

    return k


def _make_sc_degree():
    @functools.partial(
        pl.kernel,
        out_type=(jax.ShapeDtypeStruct((2, _NP, 8), jnp.float32),
                  jax.ShapeDtypeStruct((2, _NP, 8), jnp.float32)),
        mesh=_sc_mesh(),
        compiler_params=pltpu.CompilerParams(use_tc_tiling_on_sc=False),
        scratch_types=[
            pltpu.VMEM((_NBLK, _BLK), jnp.int32),
            pltpu.VMEM((_NBLK, _BLK), jnp.int32),
            pltpu.VMEM((_BLK, 8), jnp.float32),
            pltpu.VMEM_SHARED((_NP, 8), jnp.float32),
            pltpu.VMEM_SHARED((_NP, 8), jnp.float32),
            [pltpu.SemaphoreType.DMA for _ in range(8)],
            [pltpu.SemaphoreType.DMA for _ in range(8)],
        ],
    )
    def k(src_hbm, self_hbm, ones_hbm, zeros_hbm, outd_hbm, outs_hbm,
          sidx, fidx, ones, accd, accs, dsems, ssems):
        c = lax.axis_index("c")
        s = lax.axis_index("s")
        w = s * 2 + c
        pltpu.sync_copy(src_hbm.at[w], sidx)
        pltpu.sync_copy(self_hbm.at[w], fidx)
        pltpu.sync_copy(ones_hbm, ones)
        r0 = s * _RPT
        pltpu.sync_copy(zeros_hbm.at[pl.ds(r0, _RPT)], accd.at[pl.ds(r0, _RPT)])
        pltpu.sync_copy(zeros_hbm.at[pl.ds(r0, _RPT)], accs.at[pl.ds(r0, _RPT)])
        plsc.subcore_barrier()

        dd = [None] * _NBLK
        sd = [None] * _NBLK
        for j in range(_NBLK):
            b = j % _DEG_LAG
            if j >= _DEG_LAG:
                dd[j - _DEG_LAG].wait()
                sd[j - _DEG_LAG].wait()
            dd[j] = pltpu.async_copy(ones, accd.at[sidx.at[j]], dsems[b],
                                     add=True)
            sd[j] = pltpu.async_copy(ones, accs.at[fidx.at[j]], ssems[b],
                                     add=True)
        for j in range(max(0, _NBLK - _DEG_LAG), _NBLK):
            dd[j].wait()
            sd[j].wait()

        plsc.subcore_barrier()
        pltpu.sync_copy(accd.at[pl.ds(r0, _RPT)], outd_hbm.at[c].at[pl.ds(r0, _RPT)])
        pltpu.sync_copy(accs.at[pl.ds(r0, _RPT)], outs_hbm.at[c].at[pl.ds(r0, _RPT)])

    return k


def _tc_prep(degp, selfp, xp, W1, b1):
    grid = (_NP // _BR,)

    def body(degp_ref, selfp_ref, x_ref, w_ref, b_ref, dinv8_ref, g1_ref):
        deg = degp_ref[0, :, 0:1] + degp_ref[1, :, 0:1] + 1.0
        cnt = selfp_ref[0, :, 0:1] + selfp_ref[1, :, 0:1]
        dinv = lax.rsqrt(deg)
        coef = 2.0 + 3.0 * cnt
        cols = lax.broadcasted_iota(jnp.int32, (_BR, 8), 1)
        dinv8_ref[...] = jnp.where(
            cols == 0, jnp.broadcast_to(dinv, (_BR, 8)),
            jnp.where(cols == 1, jnp.broadcast_to(coef, (_BR, 8)), 0.0))
        g = jnp.dot(x_ref[...], w_ref[...],
                    preferred_element_type=jnp.float32) + b_ref[...]
        g1_ref[...] = dinv * g

    return pl.pallas_call(
        body,
        grid=grid,
        in_specs=[
            pl.BlockSpec((2, _BR, 8), lambda i: (0, i, 0)),
            pl.BlockSpec((2, _BR, 8), lambda i: (0, i, 0)),
            pl.BlockSpec((_BR, _D), lambda i: (i, 0)),
            pl.BlockSpec((_D, _NHID), lambda i: (0, 0)),
            pl.BlockSpec((1, _NHID), lambda i: (0, 0)),
        ],
        out_specs=[
            pl.BlockSpec((_BR, 8), lambda i: (i, 0)),
            pl.BlockSpec((_BR, _NHID), lambda i: (i, 0)),
        ],
        out_shape=[
            jax.ShapeDtypeStruct((_NP, 8), jnp.float32),
            jax.ShapeDtypeStruct((_NP, _NHID), jnp.float32),
        ],
    )(degp, selfp, xp, W1, b1)


def _tc_layer(accp, gp, dinv8, W, b, lap):
    F = gp.shape[1]
    F2 = W.shape[1]
    grid = (_NP // _BR,)

    def body(accp_ref, gp_ref, dinv8_ref, w_ref, b_ref, h_ref, gn_ref):
        dinv = dinv8_ref[:, 0:1]
        accsum = accp_ref[0] + accp_ref[1]
        if lap:
            coef = dinv8_ref[:, 1:2]
            h = jnp.maximum(coef * dinv * gp_ref[...] - dinv * accsum, 0.0)
        else:
            h = jnp.maximum(dinv * (accsum + gp_ref[...]), 0.0)
        h_ref[...] = h
        gn = jnp.dot(h, w_ref[...], preferred_element_type=jnp.float32) + b_ref[...]
        gn_ref[...] = dinv * gn

    return pl.pallas_call(
        body,
        grid=grid,
        in_specs=[
            pl.BlockSpec((2, _BR, F), lambda i: (0, i, 0)),
            pl.BlockSpec((_BR, F), lambda i: (i, 0)),
            pl.BlockSpec((_BR, 8), lambda i: (i, 0)),
            pl.BlockSpec((F, F2), lambda i: (0, 0)),
            pl.BlockSpec((1, F2), lambda i: (0, 0)),
        ],
        out_specs=[
            pl.BlockSpec((_BR, F), lambda i: (i, 0)),
            pl.BlockSpec((_BR, F2), lambda i: (i, 0)),
        ],
        out_shape=[
            jax.ShapeDtypeStruct((_NP, F), jnp.float32),
            jax.ShapeDtypeStruct((_NP, F2), jnp.float32),
        ],
    )(accp, gp, dinv8, W, b)


def _tc_final(accp, gp, dinv8, split=True):
    F = gp.shape[1]
    FP = accp.shape[2]
    grid = (_NP // _BR,)

    def body(accp_ref, gp_ref, dinv8_ref, out_ref):
        dinv = dinv8_ref[:, 0:1]
        coef = dinv8_ref[:, 1:2]
        if split:
            accsum = jnp.concatenate([accp_ref[0], accp_ref[1]], axis=1)
        else:
            accsum = accp_ref[0] + accp_ref[1]
        out_ref[...] = jnp.maximum(coef * dinv * gp_ref[...] - dinv * accsum, 0.0)

    return pl.pallas_call(
        body,
        grid=grid,
        in_specs=[
            pl.BlockSpec((2, _BR, FP), lambda i: (0, i, 0)),
            pl.BlockSpec((_BR, F), lambda i: (i, 0)),
            pl.BlockSpec((_BR, 8), lambda i: (i, 0)),
        ],
        out_specs=pl.BlockSpec((_BR, F), lambda i: (i, 0)),
        out_shape=jax.ShapeDtypeStruct((_NP, F), jnp.float32),
    )(accp, gp, dinv8)


@jax.jit
def kernel(x, edge_index, W1, b1, W2, b2, W3, b3, W4, b4):
    src = edge_index[0]
    dst = edge_index[1]
    pad = jnp.full((_EP - _E,), _N, jnp.int32)
    srcp = jnp.concatenate([src, pad]).reshape(_NTILES, _NBLK, _BLK)
    dstp = jnp.concatenate([dst, pad]).reshape(_NTILES, _NBLK, _BLK)
    selfi = jnp.where(src == dst, src, _N)
    selfp = jnp.concatenate([selfi, pad]).reshape(_NTILES, _NBLK, _BLK)
    xp = jnp.concatenate([x, jnp.zeros((_NP - _N, _D), jnp.float32)])

    ones8 = jnp.ones((_BLK, 8), jnp.float32)
    zeros8 = jnp.zeros((_NP, 8), jnp.float32)
    zeros16 = jnp.zeros((_NP, _LAT), jnp.float32)
    zeros32 = jnp.zeros((_NP, _NHID), jnp.float32)
    zeros64 = jnp.zeros((_NP, _D // 2), jnp.float32)

    degp, sfp = _make_sc_degree()(srcp, selfp, ones8, zeros8)
    dinv8, g1p = _tc_prep(degp, sfp, xp, W1, b1.reshape(1, -1))

    a1 = _make_sc_accum(_NHID, 8)(srcp, dstp, g1p, zeros32)
    _, g2p = _tc_layer(a1, g1p, dinv8, W2, b2.reshape(1, -1), lap=False)

    a2 = _make_sc_accum(_LAT, 8)(srcp, dstp, g2p, zeros16)
    z, g3p = _tc_layer(a2, g2p, dinv8, W3, b3.reshape(1, -1), lap=False)

    a3 = _make_sc_accum(_NHID, 8)(srcp, dstp, g3p, zeros32)
    _, g4p = _tc_layer(a3, g3p, dinv8, W4, b4.reshape(1, -1), lap=True)

    srchalf = srcp.reshape(16, 2 * _NBLK, _BLK)
    dsthalf = dstp.reshape(16, 2 * _NBLK, _BLK)
    g4h = jnp.stack([g4p[:, :_D // 2], g4p[:, _D // 2:]])
    a4 = _make_sc_accum_split(_D, 4)(srchalf, dsthalf, g4h, zeros64)
    xrec = _tc_final(a4, g4p, dinv8, split=True)

    return xrec[:_N], z[:_N]


# speedup vs baseline: 18.6926x; 1.3967x over previous
import functools

import jax
import jax.numpy as jnp
from jax import lax
from jax.experimental import pallas as pl
from jax.experimental.pallas import tpu as pltpu
from jax.experimental.pallas import tpu_sc as plsc

_N = 10000
_E = 320000
_D = 128
_NHID = 32
_LAT = 16

_NP = 10240
_NTILES = 32
_BLK = 128
_NBLK = 80
_EPT = _BLK * _NBLK
_EP = _NTILES * _EPT
_RPT = _NP // 16
_BR = 1024
_DEG_LAG = 8


def _sc_mesh():
    return plsc.VectorSubcoreMesh(core_axis_name="c", subcore_axis_name="s",
                                  num_cores=2, num_subcores=16)


def _make_sc_accum(F, nbuf):
    pf = nbuf // 2

    @functools.partial(
        pl.kernel,
        out_type=jax.ShapeDtypeStruct((2, _NP, F), jnp.float32),
        mesh=_sc_mesh(),
        compiler_params=pltpu.CompilerParams(use_tc_tiling_on_sc=False),
        scratch_types=[
            pltpu.VMEM((_NBLK, _BLK), jnp.int32),
            pltpu.VMEM((_NBLK, _BLK), jnp.int32),
            [pltpu.VMEM((_BLK, F), jnp.float32) for _ in range(nbuf)],
            pltpu.VMEM_SHARED((_NP, F), jnp.float32),
            [pltpu.SemaphoreType.DMA for _ in range(nbuf)],
            [pltpu.SemaphoreType.DMA for _ in range(nbuf)],
        ],
    )
    def k(src_hbm, dst_hbm, tbl_hbm, zeros_hbm, out_hbm,
          sidx, didx, rows, acc, gsems, ssems):
        c = lax.axis_index("c")
        s = lax.axis_index("s")
        w = s * 2 + c
        pltpu.sync_copy(src_hbm.at[w], sidx)
        pltpu.sync_copy(dst_hbm.at[w], didx)
        r0 = s * _RPT
        pltpu.sync_copy(zeros_hbm.at[pl.ds(r0, _RPT)], acc.at[pl.ds(r0, _RPT)])
        plsc.subcore_barrier()

        gd = [None] * _NBLK
        sd = [None] * _NBLK
        s_waited = [False] * _NBLK

        def fire_gather(m):
            b = m % nbuf
            gd[m] = pltpu.async_copy(tbl_hbm.at[didx.at[m]], rows[b], gsems[b])

        for m in range(pf):
            fire_gather(m)
        for j in range(_NBLK):
            b = j % nbuf
            m = j + pf
            if m < _NBLK:
                p = m - nbuf
                if p >= 0:
                    sd[p].wait()
                    s_waited[p] = True
                fire_gather(m)
            gd[j].wait()
            sd[j] = pltpu.async_copy(rows[b], acc.at[sidx.at[j]], ssems[b],
                                     add=True)
        for j in range(_NBLK):
            if not s_waited[j]:
                sd[j].wait()

        plsc.subcore_barrier()
        pltpu.sync_copy(acc.at[pl.ds(r0, _RPT)], out_hbm.at[c].at[pl.ds(r0, _RPT)])

    return k


def _make_sc_accum_split(F, nbuf):
    Fh = F // 2
    nblk = 2 * _NBLK
    pf = nbuf // 2

    @functools.partial(
        pl.kernel,
        out_type=jax.ShapeDtypeStruct((2, _NP, Fh), jnp.float32),
        mesh=_sc_mesh(),
        compiler_params=pltpu.CompilerParams(use_tc_tiling_on_sc=False),
        scratch_types=[
            pltpu.VMEM((nblk, _BLK), jnp.int32),
            pltpu.VMEM((nblk, _BLK), jnp.int32),
            [pltpu.VMEM((_BLK, Fh), jnp.float32) for _ in range(nbuf)],
            pltpu.VMEM_SHARED((_NP, Fh), jnp.float32),
            [pltpu.SemaphoreType.DMA for _ in range(nbuf)],
            [pltpu.SemaphoreType.DMA for _ in range(nbuf)],
        ],
    )
    def k(src_hbm, dst_hbm, tblh_hbm, zeros_hbm, out_hbm,
          sidx, didx, rows, acc, gsems, ssems):
        c = lax.axis_index("c")
        s = lax.axis_index("s")
        pltpu.sync_copy(src_hbm.at[s], sidx)
        pltpu.sync_copy(dst_hbm.at[s], didx)
        tbl = tblh_hbm.at[c]
        r0 = s * _RPT
        pltpu.sync_copy(zeros_hbm.at[pl.ds(r0, _RPT)], acc.at[pl.ds(r0, _RPT)])
        plsc.subcore_barrier()

        gd = [None] * nblk
        sd = [None] * nblk
        s_waited = [False] * nblk

        def fire_gather(m):
            b = m % nbuf
            gd[m] = pltpu.async_copy(tbl.at[didx.at[m]], rows[b], gsems[b])

        for m in range(pf):
            fire_gather(m)
        for j in range(nblk):
            b = j % nbuf
            m = j + pf
            if m < nblk:
                p = m - nbuf
                if p >= 0:
                    sd[p].wait()
                    s_waited[p] = True
                fire_gather(m)
            gd[j].wait()
            sd[j] = pltpu.async_copy(rows[b], acc.at[sidx.at[j]], ssems[b],
                                     add=True)
        for j in range(nblk):
            if not s_waited[j]:
                sd[j].wait()

        plsc.subcore_barrier()
        pltpu.sync_copy(acc.at[pl.ds(r0, _RPT)], out_hbm.at[c].at[pl.ds(r0, _RPT)])

    return k


def _make_sc_degree():
    @functools.partial(
        pl.kernel,
        out_type=(jax.ShapeDtypeStruct((_NTILES, _NP), jnp.float32),
                  jax.ShapeDtypeStruct((_NTILES, _NP), jnp.float32)),
        mesh=_sc_mesh(),
        compiler_params=pltpu.CompilerParams(use_tc_tiling_on_sc=False,
                                             needs_layout_passes=False),
        scratch_types=[
            pltpu.VMEM((_EPT,), jnp.int32),
            pltpu.VMEM((_EPT,), jnp.int32),
            pltpu.VMEM((_NP,), jnp.float32),
            pltpu.VMEM((_NP,), jnp.float32),
        ],
    )
    def k(src_hbm, dst_hbm, outd_hbm, outs_hbm, sidx, didx, degl, selfl):
        c = lax.axis_index("c")
        s = lax.axis_index("s")
        w = s * 2 + c
        pltpu.sync_copy(src_hbm.at[w], sidx)
        pltpu.sync_copy(dst_hbm.at[w], didx)
        zero16 = jnp.zeros((16,), jnp.float32)
        one16 = jnp.ones((16,), jnp.float32)

        @pl.loop(0, _NP // 16)
        def _(i):
            degl[pl.ds(i * 16, 16)] = zero16
            selfl[pl.ds(i * 16, 16)] = zero16

        @pl.loop(0, _EPT // 16)
        def _(i):
            s16 = sidx[pl.ds(i * 16, 16)]
            d16 = didx[pl.ds(i * 16, 16)]
            eq = jnp.where(s16 == d16, 1.0, 0.0)
            plsc.addupdate_scatter(degl, [s16], one16)
            plsc.addupdate_scatter(selfl, [s16], eq)

        pltpu.sync_copy(degl, outd_hbm.at[w])
        pltpu.sync_copy(selfl, outs_hbm.at[w])

    return k


def _tc_prep(degp, selfp, xp, W1, b1):
    grid = (_NP // _BR,)

    def body(degp_ref, selfp_ref, x_ref, w_ref, b_ref, dinv8_ref, g1_ref):
        deg = jnp.sum(degp_ref[...], axis=1, keepdims=True) + 1.0
        cnt = jnp.sum(selfp_ref[...], axis=1, keepdims=True)
        dinv = lax.rsqrt(deg)
        coef = 2.0 + 3.0 * cnt
        cols = lax.broadcasted_iota(jnp.int32, (_BR, 8), 1)
        dinv8_ref[...] = jnp.where(
            cols == 0, jnp.broadcast_to(dinv, (_BR, 8)),
            jnp.where(cols == 1, jnp.broadcast_to(coef, (_BR, 8)), 0.0))
        g = jnp.dot(x_ref[...], w_ref[...],
                    preferred_element_type=jnp.float32) + b_ref[...]
        g1_ref[...] = dinv * g

    return pl.pallas_call(
        body,
        grid=grid,
        in_specs=[
            pl.BlockSpec((_BR, _NTILES), lambda i: (i, 0)),
            pl.BlockSpec((_BR, _NTILES), lambda i: (i, 0)),
            pl.BlockSpec((_BR, _D), lambda i: (i, 0)),
            pl.BlockSpec((_D, _NHID), lambda i: (0, 0)),
            pl.BlockSpec((1, _NHID), lambda i: (0, 0)),
        ],
        out_specs=[
            pl.BlockSpec((_BR, 8), lambda i: (i, 0)),
            pl.BlockSpec((_BR, _NHID), lambda i: (i, 0)),
        ],
        out_shape=[
            jax.ShapeDtypeStruct((_NP, 8), jnp.float32),
            jax.ShapeDtypeStruct((_NP, _NHID), jnp.float32),
        ],
    )(degp, selfp, xp, W1, b1)


def _tc_layer(accp, gp, dinv8, W, b, lap):
    F = gp.shape[1]
    F2 = W.shape[1]
    grid = (_NP // _BR,)

    def body(accp_ref, gp_ref, dinv8_ref, w_ref, b_ref, h_ref, gn_ref):
        dinv = dinv8_ref[:, 0:1]
        accsum = accp_ref[0] + accp_ref[1]
        if lap:
            coef = dinv8_ref[:, 1:2]
            h = jnp.maximum(coef * dinv * gp_ref[...] - dinv * accsum, 0.0)
        else:
            h = jnp.maximum(dinv * (accsum + gp_ref[...]), 0.0)
        h_ref[...] = h
        gn = jnp.dot(h, w_ref[...], preferred_element_type=jnp.float32) + b_ref[...]
        gn_ref[...] = dinv * gn

    return pl.pallas_call(
        body,
        grid=grid,
        in_specs=[
            pl.BlockSpec((2, _BR, F), lambda i: (0, i, 0)),
            pl.BlockSpec((_BR, F), lambda i: (i, 0)),
            pl.BlockSpec((_BR, 8), lambda i: (i, 0)),
            pl.BlockSpec((F, F2), lambda i: (0, 0)),
            pl.BlockSpec((1, F2), lambda i: (0, 0)),
        ],
        out_specs=[
            pl.BlockSpec((_BR, F), lambda i: (i, 0)),
            pl.BlockSpec((_BR, F2), lambda i: (i, 0)),
        ],
        out_shape=[
            jax.ShapeDtypeStruct((_NP, F), jnp.float32),
            jax.ShapeDtypeStruct((_NP, F2), jnp.float32),
        ],
    )(accp, gp, dinv8, W, b)


def _tc_final(accp, gp, dinv8, split=True):
    F = gp.shape[1]
    FP = accp.shape[2]
    grid = (_NP // _BR,)

    def body(accp_ref, gp_ref, dinv8_ref, out_ref):
        dinv = dinv8_ref[:, 0:1]
        coef = dinv8_ref[:, 1:2]
        if split:
            accsum = jnp.concatenate([accp_ref[0], accp_ref[1]], axis=1)
        else:
            accsum = accp_ref[0] + accp_ref[1]
        out_ref[...] = jnp.maximum(coef * dinv * gp_ref[...] - dinv * accsum, 0.0)

    return pl.pallas_call(
        body,
        grid=grid,
        in_specs=[
            pl.BlockSpec((2, _BR, FP), lambda i: (0, i, 0)),
            pl.BlockSpec((_BR, F), lambda i: (i, 0)),
            pl.BlockSpec((_BR, 8), lambda i: (i, 0)),
        ],
        out_specs=pl.BlockSpec((_BR, F), lambda i: (i, 0)),
        out_shape=jax.ShapeDtypeStruct((_NP, F), jnp.float32),
    )(accp, gp, dinv8)


@jax.jit
def kernel(x, edge_index, W1, b1, W2, b2, W3, b3, W4, b4):
    src = edge_index[0]
    dst = edge_index[1]
    pad = jnp.full((_EP - _E,), _N, jnp.int32)
    srcp = jnp.concatenate([src, pad]).reshape(_NTILES, _NBLK, _BLK)
    dstp = jnp.concatenate([dst, pad]).reshape(_NTILES, _NBLK, _BLK)
    srcf = srcp.reshape(_NTILES, _EPT)
    dstf = dstp.reshape(_NTILES, _EPT)
    xp = jnp.concatenate([x, jnp.zeros((_NP - _N, _D), jnp.float32)])

    zeros16 = jnp.zeros((_NP, _LAT), jnp.float32)
    zeros32 = jnp.zeros((_NP, _NHID), jnp.float32)
    zeros64 = jnp.zeros((_NP, _D // 2), jnp.float32)

    degp, sfp = _make_sc_degree()(srcf, dstf)
    degt = jnp.transpose(degp)
    sft = jnp.transpose(sfp)
    dinv8, g1p = _tc_prep(degt, sft, xp, W1, b1.reshape(1, -1))

    a1 = _make_sc_accum(_NHID, 8)(srcp, dstp, g1p, zeros32)
    _, g2p = _tc_layer(a1, g1p, dinv8, W2, b2.reshape(1, -1), lap=False)

    a2 = _make_sc_accum(_LAT, 8)(srcp, dstp, g2p, zeros16)
    z, g3p = _tc_layer(a2, g2p, dinv8, W3, b3.reshape(1, -1), lap=False)

    a3 = _make_sc_accum(_NHID, 8)(srcp, dstp, g3p, zeros32)
    _, g4p = _tc_layer(a3, g3p, dinv8, W4, b4.reshape(1, -1), lap=True)

    srchalf = srcp.reshape(16, 2 * _NBLK, _BLK)
    dsthalf = dstp.reshape(16, 2 * _NBLK, _BLK)
    g4h = jnp.stack([g4p[:, :_D // 2], g4p[:, _D // 2:]])
    a4 = _make_sc_accum_split(_D, 4)(srchalf, dsthalf, g4h, zeros64)
    xrec = _tc_final(a4, g4p, dinv8, split=True)

    return xrec[:_N], z[:_N]
